# baseline (speedup 1.0000x reference)
import functools

import jax
import jax.numpy as jnp
from jax.experimental import pallas as pl
from jax.experimental.pallas import tpu as pltpu

_E = 8
_K = 2
_D = 1024
_H = 1024
_F = 512
_O = 1024
_N = 2048
_T = 128
_P = _N * _K + _E * _T


def _ln(t, g, b):
    m = jnp.mean(t, axis=-1, keepdims=True)
    v = jnp.mean((t - m) ** 2, axis=-1, keepdims=True)
    return (t - m) * jax.lax.rsqrt(v + 1e-5) * g + b


def _gelu(t):
    return 0.5 * t * (1.0 + jax.lax.erf(t * 0.7071067811865476))


def _expert_body(gid_ref, x_ref, w_ref,
                 fc1_ref, fc1b_ref, ln1g_ref, ln1b_ref,
                 res_ref, resb_ref,
                 fc2_ref, fc2b_ref, ln2g_ref, ln2b_ref,
                 fc3_ref, fc3b_ref, ln3g_ref, ln3b_ref,
                 fc4_ref, fc4b_ref,
                 out_ref):
    x = x_ref[...]
    h = jnp.dot(x, fc1_ref[0], preferred_element_type=jnp.float32) + fc1b_ref[0, 0]
    h = _gelu(_ln(h, ln1g_ref[0, 0], ln1b_ref[0, 0]))
    r = jnp.dot(x, res_ref[0], preferred_element_type=jnp.float32) + resb_ref[0, 0]
    h = (h + r).astype(jnp.bfloat16)
    h = jnp.dot(h, fc2_ref[0], preferred_element_type=jnp.float32) + fc2b_ref[0, 0]
    h = _gelu(_ln(h, ln2g_ref[0, 0], ln2b_ref[0, 0])).astype(jnp.bfloat16)
    h = jnp.dot(h, fc3_ref[0], preferred_element_type=jnp.float32) + fc3b_ref[0, 0]
    h = _gelu(_ln(h, ln3g_ref[0, 0], ln3b_ref[0, 0])).astype(jnp.bfloat16)
    o = jnp.dot(h, fc4_ref[0], preferred_element_type=jnp.float32) + fc4b_ref[0, 0]
    out_ref[...] = o * w_ref[:, 0:1]


def _grouped_mlp(gid, xs, ws, fc1_W, fc1_b, ln1_g, ln1_b, res_W, res_b,
                 fc2_W, fc2_b, ln2_g, ln2_b, fc3_W, fc3_b, ln3_g, ln3_b,
                 fc4_W, fc4_b):
    n_tiles = _P // _T

    def row_spec(cols):
        return pl.BlockSpec((_T, cols), lambda i, g: (i, 0))

    def w3_spec(r, c):
        return pl.BlockSpec((1, r, c), lambda i, g: (g[i], 0, 0))

    def w2_spec(c):
        return pl.BlockSpec((1, 1, c), lambda i, g: (g[i], 0, 0))

    grid_spec = pltpu.PrefetchScalarGridSpec(
        num_scalar_prefetch=1,
        grid=(n_tiles,),
        in_specs=[
            row_spec(_D),
            row_spec(128),
            w3_spec(_D, _H), w2_spec(_H), w2_spec(_H), w2_spec(_H),
            w3_spec(_D, _H), w2_spec(_H),
            w3_spec(_H, _H), w2_spec(_H), w2_spec(_H), w2_spec(_H),
            w3_spec(_H, _F), w2_spec(_F), w2_spec(_F), w2_spec(_F),
            w3_spec(_F, _O), w2_spec(_O),
        ],
        out_specs=row_spec(_O),
    )
    def v3(p):
        return p[:, None, :]

    def wb(p):
        return p.astype(jnp.bfloat16)

    return pl.pallas_call(
        _expert_body,
        grid_spec=grid_spec,
        out_shape=jax.ShapeDtypeStruct((_P, _O), jnp.float32),
    )(gid, xs, ws, wb(fc1_W), v3(fc1_b), v3(ln1_g), v3(ln1_b), wb(res_W),
      v3(res_b), wb(fc2_W), v3(fc2_b), v3(ln2_g), v3(ln2_b), wb(fc3_W),
      v3(fc3_b), v3(ln3_g), v3(ln3_b), wb(fc4_W), v3(fc4_b))


def kernel(x, gate_W, gate_b, fc1_W, fc1_b, ln1_g, ln1_b, res_W, res_b,
           fc2_W, fc2_b, ln2_g, ln2_b, fc3_W, fc3_b, ln3_g, ln3_b,
           fc4_W, fc4_b):
    logits = x @ gate_W + gate_b
    probs = jax.nn.softmax(logits, axis=-1)
    topv, topi = jax.lax.top_k(probs, _K)
    wn = topv / (jnp.sum(topv, axis=-1, keepdims=True) + 1e-9)

    e_flat = topi.reshape(-1).astype(jnp.int32)
    onehot = (e_flat[:, None] == jnp.arange(_E, dtype=jnp.int32)[None, :])
    csum = jnp.cumsum(onehot.astype(jnp.int32), axis=0)
    rank = jnp.take_along_axis(csum, e_flat[:, None], axis=1)[:, 0] - 1
    counts = csum[-1]

    padded = ((counts + _T - 1) // _T) * _T
    pad_end = jnp.cumsum(padded)
    pad_off = pad_end - padded
    dest = (pad_off[e_flat] + rank).astype(jnp.int32)

    tok = (jnp.arange(_N * _K, dtype=jnp.int32) // _K)
    gather_idx = jnp.zeros((_P,), jnp.int32).at[dest].set(tok)
    w_arr = jnp.zeros((_P,), jnp.float32).at[dest].set(wn.reshape(-1))
    pos = dest.reshape(_N, _K)

    tile_start = jnp.arange(_P // _T, dtype=jnp.int32) * _T
    gid = jnp.searchsorted(pad_end, tile_start, side='right')
    gid = jnp.minimum(gid, _E - 1).astype(jnp.int32)

    xb = x.astype(jnp.bfloat16)
    xs = jnp.take(xb, gather_idx, axis=0)
    ws = jnp.broadcast_to(w_arr[:, None], (_P, 128))

    out_sorted = _grouped_mlp(
        gid, xs, ws, fc1_W, fc1_b, ln1_g, ln1_b, res_W, res_b,
        fc2_W, fc2_b, ln2_g, ln2_b, fc3_W, fc3_b, ln3_g, ln3_b,
        fc4_W, fc4_b)

    y = jnp.take(out_sorted, pos[:, 0], axis=0) + jnp.take(out_sorted, pos[:, 1], axis=0)
    return y


# speedup vs baseline: 1.0061x; 1.0061x over previous
import functools

import jax
import jax.numpy as jnp
from jax.experimental import pallas as pl
from jax.experimental.pallas import tpu as pltpu

_E = 8
_K = 2
_D = 1024
_H = 1024
_F = 512
_O = 1024
_N = 2048
_T = 128
_P = _N * _K + _E * _T
_V = 9728


def _ln(t, g, b):
    m = jnp.mean(t, axis=-1, keepdims=True)
    v = jnp.mean((t - m) ** 2, axis=-1, keepdims=True)
    return (t - m) * jax.lax.rsqrt(v + 1e-5) * g + b


def _gelu(t):
    return 0.5 * t * (1.0 + jax.lax.erf(t * 0.7071067811865476))


def _expert_body(gid_ref, x_ref, w1r_ref, w2_ref, w3_ref, w4_ref, vec_ref,
                 out_ref):
    x = x_ref[...]
    v = vec_ref[0, 0]
    hr = jnp.dot(x, w1r_ref[0], preferred_element_type=jnp.float32)
    h = hr[:, :_H] + v[0:1024]
    r = hr[:, _H:] + v[1024:2048]
    h = _gelu(_ln(h, v[2048:3072], v[3072:4096])) + r
    h = h.astype(jnp.bfloat16)
    h = jnp.dot(h, w2_ref[0], preferred_element_type=jnp.float32) + v[4096:5120]
    h = _gelu(_ln(h, v[5120:6144], v[6144:7168])).astype(jnp.bfloat16)
    h = jnp.dot(h, w3_ref[0], preferred_element_type=jnp.float32) + v[7168:7680]
    h = _gelu(_ln(h, v[7680:8192], v[8192:8704])).astype(jnp.bfloat16)
    o = jnp.dot(h, w4_ref[0], preferred_element_type=jnp.float32) + v[8704:9728]
    out_ref[...] = o.astype(jnp.bfloat16)


def _grouped_mlp(gid, xs, w1r, w2, w3, w4, vec):
    n_tiles = _P // _T

    def row_spec(cols):
        return pl.BlockSpec((_T, cols), lambda i, g: (i, 0))

    def w3_spec(r, c):
        return pl.BlockSpec((1, r, c), lambda i, g: (g[i], 0, 0))

    grid_spec = pltpu.PrefetchScalarGridSpec(
        num_scalar_prefetch=1,
        grid=(n_tiles,),
        in_specs=[
            row_spec(_D),
            w3_spec(_D, 2 * _H),
            w3_spec(_H, _H),
            w3_spec(_H, _F),
            w3_spec(_F, _O),
            pl.BlockSpec((1, 1, _V), lambda i, g: (g[i], 0, 0)),
        ],
        out_specs=row_spec(_O),
    )
    return pl.pallas_call(
        _expert_body,
        grid_spec=grid_spec,
        out_shape=jax.ShapeDtypeStruct((_P, _O), jnp.bfloat16),
    )(gid, xs, w1r, w2, w3, w4, vec)


def kernel(x, gate_W, gate_b, fc1_W, fc1_b, ln1_g, ln1_b, res_W, res_b,
           fc2_W, fc2_b, ln2_g, ln2_b, fc3_W, fc3_b, ln3_g, ln3_b,
           fc4_W, fc4_b):
    logits = x @ gate_W + gate_b
    probs = jax.nn.softmax(logits, axis=-1)
    topv, topi = jax.lax.top_k(probs, _K)
    wn = topv / (jnp.sum(topv, axis=-1, keepdims=True) + 1e-9)

    e_flat = topi.reshape(-1).astype(jnp.int32)
    onehot = (e_flat[:, None] == jnp.arange(_E, dtype=jnp.int32)[None, :])
    csum = jnp.cumsum(onehot.astype(jnp.int32), axis=0)
    rank = jnp.take_along_axis(csum, e_flat[:, None], axis=1)[:, 0] - 1
    counts = csum[-1]

    padded = ((counts + _T - 1) // _T) * _T
    pad_end = jnp.cumsum(padded)
    pad_off = pad_end - padded
    dest = (pad_off[e_flat] + rank).astype(jnp.int32)

    tok = (jnp.arange(_N * _K, dtype=jnp.int32) // _K)
    gather_idx = jnp.zeros((_P,), jnp.int32).at[dest].set(tok)
    pos = dest.reshape(_N, _K)

    tile_start = jnp.arange(_P // _T, dtype=jnp.int32) * _T
    gid = jnp.searchsorted(pad_end, tile_start, side='right')
    gid = jnp.minimum(gid, _E - 1).astype(jnp.int32)

    xs = jnp.take(x.astype(jnp.bfloat16), gather_idx, axis=0)

    w1r = jnp.concatenate([fc1_W, res_W], axis=2).astype(jnp.bfloat16)
    vec = jnp.concatenate(
        [fc1_b, res_b, ln1_g, ln1_b, fc2_b, ln2_g, ln2_b,
         fc3_b, ln3_g, ln3_b, fc4_b], axis=1)[:, None, :]

    out_sorted = _grouped_mlp(
        gid, xs, w1r, fc2_W.astype(jnp.bfloat16), fc3_W.astype(jnp.bfloat16),
        fc4_W.astype(jnp.bfloat16), vec)

    y = (jnp.take(out_sorted, pos[:, 0], axis=0) * wn[:, 0:1]
         + jnp.take(out_sorted, pos[:, 1], axis=0) * wn[:, 1:2])
    return y


# speedup vs baseline: 1.1166x; 1.1099x over previous
import functools

import jax
import jax.numpy as jnp
from jax.experimental import pallas as pl
from jax.experimental.pallas import tpu as pltpu

_E = 8
_K = 2
_D = 1024
_H = 1024
_F = 512
_O = 1024
_N = 2048
_T = 256
_P = _N * _K + _E * _T
_V = 9728


def _ln(t, g, b):
    m = jnp.mean(t, axis=-1, keepdims=True)
    v = jnp.mean((t - m) ** 2, axis=-1, keepdims=True)
    return (t - m) * jax.lax.rsqrt(v + 1e-5) * g + b


def _gelu(t):
    return 0.5 * t * (1.0 + jax.lax.erf(t * 0.7071067811865476))


def _expert_body(gid_ref, x_ref, w1r_ref, w2_ref, w3_ref, w4_ref, vec_ref,
                 out_ref):
    x = x_ref[...]
    v = vec_ref[0, 0]
    hr = jnp.dot(x, w1r_ref[0], preferred_element_type=jnp.float32)
    h = hr[:, :_H] + v[0:1024]
    r = hr[:, _H:] + v[1024:2048]
    h = _gelu(_ln(h, v[2048:3072], v[3072:4096])) + r
    h = h.astype(jnp.bfloat16)
    h = jnp.dot(h, w2_ref[0], preferred_element_type=jnp.float32) + v[4096:5120]
    h = _gelu(_ln(h, v[5120:6144], v[6144:7168])).astype(jnp.bfloat16)
    h = jnp.dot(h, w3_ref[0], preferred_element_type=jnp.float32) + v[7168:7680]
    h = _gelu(_ln(h, v[7680:8192], v[8192:8704])).astype(jnp.bfloat16)
    o = jnp.dot(h, w4_ref[0], preferred_element_type=jnp.float32) + v[8704:9728]
    out_ref[...] = o.astype(jnp.bfloat16)


def _grouped_mlp(gid, xs, w1r, w2, w3, w4, vec):
    n_tiles = _P // _T

    def row_spec(cols):
        return pl.BlockSpec((_T, cols), lambda i, g: (i, 0))

    def w3_spec(r, c):
        return pl.BlockSpec((1, r, c), lambda i, g: (g[i], 0, 0))

    grid_spec = pltpu.PrefetchScalarGridSpec(
        num_scalar_prefetch=1,
        grid=(n_tiles,),
        in_specs=[
            row_spec(_D),
            w3_spec(_D, 2 * _H),
            w3_spec(_H, _H),
            w3_spec(_H, _F),
            w3_spec(_F, _O),
            pl.BlockSpec((1, 1, _V), lambda i, g: (g[i], 0, 0)),
        ],
        out_specs=row_spec(_O),
    )
    return pl.pallas_call(
        _expert_body,
        grid_spec=grid_spec,
        out_shape=jax.ShapeDtypeStruct((_P, _O), jnp.bfloat16),
    )(gid, xs, w1r, w2, w3, w4, vec)


def kernel(x, gate_W, gate_b, fc1_W, fc1_b, ln1_g, ln1_b, res_W, res_b,
           fc2_W, fc2_b, ln2_g, ln2_b, fc3_W, fc3_b, ln3_g, ln3_b,
           fc4_W, fc4_b):
    logits = x @ gate_W + gate_b
    probs = jax.nn.softmax(logits, axis=-1)
    topv, topi = jax.lax.top_k(probs, _K)
    wn = topv / (jnp.sum(topv, axis=-1, keepdims=True) + 1e-9)

    e_flat = topi.reshape(-1).astype(jnp.int32)
    onehot = (e_flat[:, None] == jnp.arange(_E, dtype=jnp.int32)[None, :])
    csum = jnp.cumsum(onehot.astype(jnp.int32), axis=0)
    rank = jnp.take_along_axis(csum, e_flat[:, None], axis=1)[:, 0] - 1
    counts = csum[-1]

    padded = ((counts + _T - 1) // _T) * _T
    pad_end = jnp.cumsum(padded)
    pad_off = pad_end - padded
    dest = (pad_off[e_flat] + rank).astype(jnp.int32)

    tok = (jnp.arange(_N * _K, dtype=jnp.int32) // _K)
    gather_idx = jnp.zeros((_P,), jnp.int32).at[dest].set(tok)
    pos = dest.reshape(_N, _K)

    tile_start = jnp.arange(_P // _T, dtype=jnp.int32) * _T
    gid = jnp.searchsorted(pad_end, tile_start, side='right')
    gid = jnp.minimum(gid, _E - 1).astype(jnp.int32)

    xs = jnp.take(x.astype(jnp.bfloat16), gather_idx, axis=0)

    w1r = jnp.concatenate([fc1_W, res_W], axis=2).astype(jnp.bfloat16)
    vec = jnp.concatenate(
        [fc1_b, res_b, ln1_g, ln1_b, fc2_b, ln2_g, ln2_b,
         fc3_b, ln3_g, ln3_b, fc4_b], axis=1)[:, None, :]

    out_sorted = _grouped_mlp(
        gid, xs, w1r, fc2_W.astype(jnp.bfloat16), fc3_W.astype(jnp.bfloat16),
        fc4_W.astype(jnp.bfloat16), vec)

    y = (jnp.take(out_sorted, pos[:, 0], axis=0) * wn[:, 0:1]
         + jnp.take(out_sorted, pos[:, 1], axis=0) * wn[:, 1:2])
    return y
